# baseline (speedup 1.0000x reference)
import functools

import jax
import jax.numpy as jnp
from jax import lax
from jax.experimental import pallas as pl
from jax.experimental.pallas import tpu as pltpu
from jax.experimental.pallas import tpu_sc as plsc

F32 = jnp.float32

_N = 10000
_E = 320000
_D = 128
_H = 128
_C = 64

_CH = 64
_NSC = 2
_NTS = 16
_NW = _NSC * _NTS
_RPW = 158
_ROWS = _RPW * _NW
_EP = _ROWS * _CH
_SPILL = 8
_BPT = 624
_LAST = _N - _BPT * _NTS

_R = 1000
_G = _N // _R


def _edge_agg(table, src2d, dst2d, W):
    mesh = plsc.VectorSubcoreMesh(core_axis_name="c", subcore_axis_name="s")

    def body(tbl, src_h, dst_h, out_a, out_b, src_v, dst_v, rows_v, rows_w, acc,
             sem, semb, sems, semsb):
        c = lax.axis_index("c")
        s = lax.axis_index("s")
        wid = c * _NTS + s

        def zb(i, carry):
            r = i // (W // 16)
            col = (i % (W // 16)) * 16
            rows_v[r, pl.ds(col, 16)] = jnp.zeros((16,), F32)
            return carry
        lax.fori_loop(0, _CH * (W // 16), zb, 0)
        row0 = s * _BPT
        for k in range(_BPT // _CH):
            pltpu.sync_copy(rows_v, acc.at[pl.ds(row0 + k * _CH, _CH)])
        if _BPT % _CH:
            pltpu.sync_copy(rows_v.at[pl.ds(0, _BPT % _CH)],
                            acc.at[pl.ds(row0 + (_BPT // _CH) * _CH, _BPT % _CH)])

        @pl.when(s == _NTS - 1)
        def _():
            pltpu.sync_copy(rows_v.at[pl.ds(0, _LAST)],
                            acc.at[pl.ds(_NTS * _BPT, _LAST)])
        plsc.subcore_barrier()

        rb = wid * _RPW
        pltpu.sync_copy(src_h.at[pl.ds(rb, _RPW)], src_v)
        pltpu.sync_copy(dst_h.at[pl.ds(rb, _RPW)], dst_v)

        pltpu.async_copy(tbl.at[src_v.at[0]], rows_v, sem)
        pltpu.async_copy(tbl.at[src_v.at[1]], rows_w, semb)

        def chunk(jj, carry):
            j = 2 * jj
            pltpu.make_async_copy(tbl.at[src_v.at[j]], rows_v, sem).wait()
            s_a = pltpu.async_copy(rows_v, acc.at[dst_v.at[j]], sems, add=True)
            pltpu.make_async_copy(tbl.at[src_v.at[j + 1]], rows_w, semb).wait()
            s_b = pltpu.async_copy(rows_w, acc.at[dst_v.at[j + 1]], semsb, add=True)
            s_a.wait()

            @pl.when(jj < _RPW // 2 - 1)
            def _():
                pltpu.async_copy(tbl.at[src_v.at[j + 2]], rows_v, sem)
            s_b.wait()

            @pl.when(jj < _RPW // 2 - 1)
            def _():
                pltpu.async_copy(tbl.at[src_v.at[j + 3]], rows_w, semb)
            return carry
        lax.fori_loop(0, _RPW // 2, chunk, 0)
        plsc.subcore_barrier()

        @pl.when(c == 0)
        def _():
            pltpu.sync_copy(acc.at[pl.ds(row0, _BPT)], out_a.at[pl.ds(row0, _BPT)])

            @pl.when(s == _NTS - 1)
            def _():
                pltpu.sync_copy(acc.at[pl.ds(_NTS * _BPT, _LAST)],
                                out_a.at[pl.ds(_NTS * _BPT, _LAST)])

        @pl.when(c == 1)
        def _():
            pltpu.sync_copy(acc.at[pl.ds(row0, _BPT)], out_b.at[pl.ds(row0, _BPT)])

            @pl.when(s == _NTS - 1)
            def _():
                pltpu.sync_copy(acc.at[pl.ds(_NTS * _BPT, _LAST)],
                                out_b.at[pl.ds(_NTS * _BPT, _LAST)])

    out = jax.ShapeDtypeStruct((_N, W), F32)
    k = pl.kernel(
        body,
        out_type=(out, out),
        mesh=mesh,
        compiler_params=pltpu.CompilerParams(use_tc_tiling_on_sc=False),
        scratch_types=[
            pltpu.VMEM((_RPW, _CH), jnp.int32),
            pltpu.VMEM((_RPW, _CH), jnp.int32),
            pltpu.VMEM((_CH, W), F32),
            pltpu.VMEM((_CH, W), F32),
            pltpu.VMEM_SHARED((_N + _SPILL, W), F32),
            pltpu.SemaphoreType.DMA,
            pltpu.SemaphoreType.DMA,
            pltpu.SemaphoreType.DMA,
            pltpu.SemaphoreType.DMA,
        ],
    )
    return k(table, src2d, dst2d)


def _mm_aug(x, w1n):
    def body(x_ref, w_ref, o_ref):
        mm = jnp.dot(x_ref[...], w_ref[...], preferred_element_type=F32)
        o_ref[...] = jnp.concatenate([mm, jnp.ones((_R, 16), F32)], axis=1)

    return pl.pallas_call(
        body,
        grid=(_G,),
        in_specs=[
            pl.BlockSpec((_R, _D), lambda i: (i, 0)),
            pl.BlockSpec((_D, _H), lambda i: (0, 0)),
        ],
        out_specs=pl.BlockSpec((_R, _H + 16), lambda i: (i, 0)),
        out_shape=jax.ShapeDtypeStruct((_N, _H + 16), F32),
    )(x, w1n)


def _combine1(x, w1s, b1, acc_a, acc_b, w2n):
    def body(x_ref, ws_ref, b_ref, aa_ref, ab_ref, wn_ref, h2_ref, hn2_ref, iv_ref):
        aa = aa_ref[...]
        ab = ab_ref[...]
        deg = aa[:, _H:_H + 16] + ab[:, _H:_H + 16]
        invd = 1.0 / jnp.maximum(deg, 1.0)
        agg = aa[:, 0:_H] + ab[:, 0:_H]
        mean = agg * invd[:, 0:1]
        h2 = jnp.maximum(
            jnp.dot(x_ref[...], ws_ref[...], preferred_element_type=F32)
            + b_ref[...] + mean, 0.0)
        h2_ref[...] = h2
        hn2_ref[...] = jnp.dot(h2, wn_ref[...], preferred_element_type=F32)
        iv_ref[...] = invd

    return pl.pallas_call(
        body,
        grid=(_G,),
        in_specs=[
            pl.BlockSpec((_R, _D), lambda i: (i, 0)),
            pl.BlockSpec((_D, _H), lambda i: (0, 0)),
            pl.BlockSpec((1, _H), lambda i: (0, 0)),
            pl.BlockSpec((_R, _H + 16), lambda i: (i, 0)),
            pl.BlockSpec((_R, _H + 16), lambda i: (i, 0)),
            pl.BlockSpec((_H, _H), lambda i: (0, 0)),
        ],
        out_specs=[
            pl.BlockSpec((_R, _H), lambda i: (i, 0)),
            pl.BlockSpec((_R, _H), lambda i: (i, 0)),
            pl.BlockSpec((_R, 16), lambda i: (i, 0)),
        ],
        out_shape=[
            jax.ShapeDtypeStruct((_N, _H), F32),
            jax.ShapeDtypeStruct((_N, _H), F32),
            jax.ShapeDtypeStruct((_N, 16), F32),
        ],
    )(x, w1s, b1.reshape(1, _H), acc_a, acc_b, w2n)


def _combine2(h2, w2s, b2, acc_a, acc_b, invd, w3n):
    def body(h_ref, ws_ref, b_ref, aa_ref, ab_ref, iv_ref, wn_ref, h3_ref, hn3_ref):
        mean = (aa_ref[...] + ab_ref[...]) * iv_ref[...][:, 0:1]
        h3 = jnp.maximum(
            jnp.dot(h_ref[...], ws_ref[...], preferred_element_type=F32)
            + b_ref[...] + mean, 0.0)
        h3_ref[...] = h3
        hn3_ref[...] = jnp.dot(h3, wn_ref[...], preferred_element_type=F32)

    return pl.pallas_call(
        body,
        grid=(_G,),
        in_specs=[
            pl.BlockSpec((_R, _H), lambda i: (i, 0)),
            pl.BlockSpec((_H, _H), lambda i: (0, 0)),
            pl.BlockSpec((1, _H), lambda i: (0, 0)),
            pl.BlockSpec((_R, _H), lambda i: (i, 0)),
            pl.BlockSpec((_R, _H), lambda i: (i, 0)),
            pl.BlockSpec((_R, 16), lambda i: (i, 0)),
            pl.BlockSpec((_H, _C), lambda i: (0, 0)),
        ],
        out_specs=[
            pl.BlockSpec((_R, _H), lambda i: (i, 0)),
            pl.BlockSpec((_R, _C), lambda i: (i, 0)),
        ],
        out_shape=[
            jax.ShapeDtypeStruct((_N, _H), F32),
            jax.ShapeDtypeStruct((_N, _C), F32),
        ],
    )(h2, w2s, b2.reshape(1, _H), acc_a, acc_b, invd, w3n)


def _final(h3, w3s, b3, acc_a, acc_b, invd):
    def body(h_ref, ws_ref, b_ref, aa_ref, ab_ref, iv_ref, o_ref):
        mean = (aa_ref[...] + ab_ref[...]) * iv_ref[...][:, 0:1]
        z = jnp.maximum(
            jnp.dot(h_ref[...], ws_ref[...], preferred_element_type=F32)
            + b_ref[...] + mean, 0.0)
        m = jnp.max(z, axis=1, keepdims=True)
        lse = jnp.log(jnp.sum(jnp.exp(z - m), axis=1, keepdims=True)) + m
        o_ref[...] = z - lse

    return pl.pallas_call(
        body,
        grid=(_G,),
        in_specs=[
            pl.BlockSpec((_R, _H), lambda i: (i, 0)),
            pl.BlockSpec((_H, _C), lambda i: (0, 0)),
            pl.BlockSpec((1, _C), lambda i: (0, 0)),
            pl.BlockSpec((_R, _C), lambda i: (i, 0)),
            pl.BlockSpec((_R, _C), lambda i: (i, 0)),
            pl.BlockSpec((_R, 16), lambda i: (i, 0)),
        ],
        out_specs=pl.BlockSpec((_R, _C), lambda i: (i, 0)),
        out_shape=jax.ShapeDtypeStruct((_N, _C), F32),
    )(h3, w3s, b3.reshape(1, _C), acc_a, acc_b, invd)


def kernel(x, edge_index, W1s, W1n, b1, W2s, W2n, b2, W3s, W3n, b3):
    npad = _EP - _E
    src2d = jnp.concatenate(
        [edge_index[0], jnp.zeros((npad,), jnp.int32)]).reshape(_ROWS, _CH)
    dst2d = jnp.concatenate(
        [edge_index[1], jnp.full((npad,), _N, jnp.int32)]).reshape(_ROWS, _CH)

    hn1 = _mm_aug(x, W1n)
    a1a, a1b = _edge_agg(hn1, src2d, dst2d, _H + 16)
    h2, hn2, invd = _combine1(x, W1s, b1, a1a, a1b, W2n)
    a2a, a2b = _edge_agg(hn2, src2d, dst2d, _H)
    h3, hn3 = _combine2(h2, W2s, b2, a2a, a2b, invd, W3n)
    a3a, a3b = _edge_agg(hn3, src2d, dst2d, _C)
    return _final(h3, W3s, b3, a3a, a3b, invd)


# speedup vs baseline: 1.1033x; 1.1033x over previous
import jax
import jax.numpy as jnp
from jax import lax
from jax.experimental import pallas as pl
from jax.experimental.pallas import tpu as pltpu
from jax.experimental.pallas import tpu_sc as plsc

F32 = jnp.float32

_N = 10000
_E = 320000
_D = 128
_H = 128
_C = 64

_CH = 64
_NSC = 2
_NTS = 16
_NW = _NSC * _NTS
_RPW = 158
_ROWS = _RPW * _NW
_EP = _ROWS * _CH
_SPILL = 8
_BPT = 624
_LAST = _N - _BPT * _NTS
_DW = 16

_R = 1000
_G = _N // _R


def _edge_agg(table, src2d, dst2d, W, with_deg=False):
    mesh = plsc.VectorSubcoreMesh(core_axis_name="c", subcore_axis_name="s")

    def body(tbl, src_h, dst_h, *rest):
        if with_deg:
            (out_a, out_b, deg_a, deg_b,
             src_v, dst_v, rows_v, rows_w, ones_v, acc, dacc, sem, semb) = rest
        else:
            (out_a, out_b,
             src_v, dst_v, rows_v, rows_w, acc, sem, semb) = rest
        c = lax.axis_index("c")
        s = lax.axis_index("s")
        wid = c * _NTS + s

        def zb(i, carry):
            r = i // (W // 16)
            col = (i % (W // 16)) * 16
            rows_v[r, pl.ds(col, 16)] = jnp.zeros((16,), F32)
            return carry
        lax.fori_loop(0, _CH * (W // 16), zb, 0)
        row0 = s * _BPT
        for k in range(_BPT // _CH):
            pltpu.sync_copy(rows_v, acc.at[pl.ds(row0 + k * _CH, _CH)])
        if _BPT % _CH:
            pltpu.sync_copy(rows_v.at[pl.ds(0, _BPT % _CH)],
                            acc.at[pl.ds(row0 + (_BPT // _CH) * _CH, _BPT % _CH)])

        @pl.when(s == _NTS - 1)
        def _():
            pltpu.sync_copy(rows_v.at[pl.ds(0, _LAST)],
                            acc.at[pl.ds(_NTS * _BPT, _LAST)])

        if with_deg:
            def zo(i, carry):
                ones_v[i, pl.ds(0, _DW)] = jnp.zeros((_DW,), F32)
                return carry
            lax.fori_loop(0, _CH, zo, 0)
            for k in range(_BPT // _CH):
                pltpu.sync_copy(ones_v, dacc.at[pl.ds(row0 + k * _CH, _CH)])
            if _BPT % _CH:
                pltpu.sync_copy(ones_v.at[pl.ds(0, _BPT % _CH)],
                                dacc.at[pl.ds(row0 + (_BPT // _CH) * _CH,
                                              _BPT % _CH)])

            @pl.when(s == _NTS - 1)
            def _():
                pltpu.sync_copy(ones_v.at[pl.ds(0, _LAST)],
                                dacc.at[pl.ds(_NTS * _BPT, _LAST)])

            def fo(i, carry):
                ones_v[i, pl.ds(0, _DW)] = jnp.full((_DW,), 1.0, F32)
                return carry
            lax.fori_loop(0, _CH, fo, 0)
        plsc.subcore_barrier()

        rb = wid * _RPW
        pltpu.sync_copy(src_h.at[pl.ds(rb, _RPW)], src_v)
        pltpu.sync_copy(dst_h.at[pl.ds(rb, _RPW)], dst_v)

        pltpu.async_copy(tbl.at[src_v.at[0]], rows_v, sem)

        def chunk(jj, carry):
            j = 2 * jj
            cpb = pltpu.async_copy(tbl.at[src_v.at[j + 1]], rows_w, semb)
            pltpu.make_async_copy(tbl.at[src_v.at[j]], rows_v, sem).wait()
            pltpu.sync_copy(rows_v, acc.at[dst_v.at[j]], add=True)
            if with_deg:
                pltpu.sync_copy(ones_v, dacc.at[dst_v.at[j]], add=True)

            @pl.when(jj < _RPW // 2 - 1)
            def _():
                pltpu.async_copy(tbl.at[src_v.at[j + 2]], rows_v, sem)
            cpb.wait()
            pltpu.sync_copy(rows_w, acc.at[dst_v.at[j + 1]], add=True)
            if with_deg:
                pltpu.sync_copy(ones_v, dacc.at[dst_v.at[j + 1]], add=True)
            return carry
        lax.fori_loop(0, _RPW // 2, chunk, 0)
        plsc.subcore_barrier()

        def copy_out(dst_hbm, deg_hbm):
            pltpu.sync_copy(acc.at[pl.ds(row0, _BPT)],
                            dst_hbm.at[pl.ds(row0, _BPT)])
            if with_deg:
                pltpu.sync_copy(dacc.at[pl.ds(row0, _BPT)],
                                deg_hbm.at[pl.ds(row0, _BPT)])

            @pl.when(s == _NTS - 1)
            def _():
                pltpu.sync_copy(acc.at[pl.ds(_NTS * _BPT, _LAST)],
                                dst_hbm.at[pl.ds(_NTS * _BPT, _LAST)])
                if with_deg:
                    pltpu.sync_copy(dacc.at[pl.ds(_NTS * _BPT, _LAST)],
                                    deg_hbm.at[pl.ds(_NTS * _BPT, _LAST)])

        @pl.when(c == 0)
        def _():
            copy_out(out_a, deg_a if with_deg else None)

        @pl.when(c == 1)
        def _():
            copy_out(out_b, deg_b if with_deg else None)

    out = jax.ShapeDtypeStruct((_N, W), F32)
    deg = jax.ShapeDtypeStruct((_N, _DW), F32)
    out_type = (out, out, deg, deg) if with_deg else (out, out)
    scratch = [
        pltpu.VMEM((_RPW, _CH), jnp.int32),
        pltpu.VMEM((_RPW, _CH), jnp.int32),
        pltpu.VMEM((_CH, W), F32),
        pltpu.VMEM((_CH, W), F32),
    ]
    if with_deg:
        scratch.append(pltpu.VMEM((_CH, _DW), F32))
    scratch.append(pltpu.VMEM_SHARED((_N + _SPILL, W), F32))
    if with_deg:
        scratch.append(pltpu.VMEM_SHARED((_N + _SPILL, _DW), F32))
    scratch += [pltpu.SemaphoreType.DMA, pltpu.SemaphoreType.DMA]
    k = pl.kernel(
        body,
        out_type=out_type,
        mesh=mesh,
        compiler_params=pltpu.CompilerParams(use_tc_tiling_on_sc=False),
        scratch_types=scratch,
    )
    return k(table, src2d, dst2d)


def _pre(h, w, b, dout):
    din = h.shape[1]

    def body(h_ref, w_ref, b_ref, o_ref):
        o_ref[...] = (jnp.dot(h_ref[...], w_ref[...], preferred_element_type=F32)
                      + b_ref[...])

    return pl.pallas_call(
        body,
        grid=(_G,),
        in_specs=[
            pl.BlockSpec((_R, din), lambda i: (i, 0)),
            pl.BlockSpec((din, dout), lambda i: (0, 0)),
            pl.BlockSpec((1, dout), lambda i: (0, 0)),
        ],
        out_specs=pl.BlockSpec((_R, dout), lambda i: (i, 0)),
        out_shape=jax.ShapeDtypeStruct((_N, dout), F32),
    )(h, w, b.reshape(1, dout))


def _combine1(pre1, acc_a, acc_b, deg_a, deg_b, w1n, w2n):
    def body(p_ref, aa_ref, ab_ref, da_ref, db_ref, wn_ref, w2_ref,
             h2_ref, hn2_ref, iv_ref):
        deg = da_ref[...] + db_ref[...]
        invd = 1.0 / jnp.maximum(deg, 1.0)
        mean = (aa_ref[...] + ab_ref[...]) * invd[:, 0:1]
        h2 = jnp.maximum(
            p_ref[...] + jnp.dot(mean, wn_ref[...], preferred_element_type=F32),
            0.0)
        h2_ref[...] = h2
        hn2_ref[...] = jnp.dot(h2, w2_ref[...], preferred_element_type=F32)
        iv_ref[...] = invd

    return pl.pallas_call(
        body,
        grid=(_G,),
        in_specs=[
            pl.BlockSpec((_R, _H), lambda i: (i, 0)),
            pl.BlockSpec((_R, _D), lambda i: (i, 0)),
            pl.BlockSpec((_R, _D), lambda i: (i, 0)),
            pl.BlockSpec((_R, _DW), lambda i: (i, 0)),
            pl.BlockSpec((_R, _DW), lambda i: (i, 0)),
            pl.BlockSpec((_D, _H), lambda i: (0, 0)),
            pl.BlockSpec((_H, _H), lambda i: (0, 0)),
        ],
        out_specs=[
            pl.BlockSpec((_R, _H), lambda i: (i, 0)),
            pl.BlockSpec((_R, _H), lambda i: (i, 0)),
            pl.BlockSpec((_R, _DW), lambda i: (i, 0)),
        ],
        out_shape=[
            jax.ShapeDtypeStruct((_N, _H), F32),
            jax.ShapeDtypeStruct((_N, _H), F32),
            jax.ShapeDtypeStruct((_N, _DW), F32),
        ],
    )(pre1, acc_a, acc_b, deg_a, deg_b, w1n, w2n)


def _combine2(pre2, acc_a, acc_b, invd, w3n):
    def body(p_ref, aa_ref, ab_ref, iv_ref, wn_ref, h3_ref, hn3_ref):
        mean = (aa_ref[...] + ab_ref[...]) * iv_ref[...][:, 0:1]
        h3 = jnp.maximum(p_ref[...] + mean, 0.0)
        h3_ref[...] = h3
        hn3_ref[...] = jnp.dot(h3, wn_ref[...], preferred_element_type=F32)

    return pl.pallas_call(
        body,
        grid=(_G,),
        in_specs=[
            pl.BlockSpec((_R, _H), lambda i: (i, 0)),
            pl.BlockSpec((_R, _H), lambda i: (i, 0)),
            pl.BlockSpec((_R, _H), lambda i: (i, 0)),
            pl.BlockSpec((_R, _DW), lambda i: (i, 0)),
            pl.BlockSpec((_H, _C), lambda i: (0, 0)),
        ],
        out_specs=[
            pl.BlockSpec((_R, _H), lambda i: (i, 0)),
            pl.BlockSpec((_R, _C), lambda i: (i, 0)),
        ],
        out_shape=[
            jax.ShapeDtypeStruct((_N, _H), F32),
            jax.ShapeDtypeStruct((_N, _C), F32),
        ],
    )(pre2, acc_a, acc_b, invd, w3n)


def _final(pre3, acc_a, acc_b, invd):
    def body(p_ref, aa_ref, ab_ref, iv_ref, o_ref):
        mean = (aa_ref[...] + ab_ref[...]) * iv_ref[...][:, 0:1]
        z = jnp.maximum(p_ref[...] + mean, 0.0)
        m = jnp.max(z, axis=1, keepdims=True)
        lse = jnp.log(jnp.sum(jnp.exp(z - m), axis=1, keepdims=True)) + m
        o_ref[...] = z - lse

    return pl.pallas_call(
        body,
        grid=(_G,),
        in_specs=[
            pl.BlockSpec((_R, _C), lambda i: (i, 0)),
            pl.BlockSpec((_R, _C), lambda i: (i, 0)),
            pl.BlockSpec((_R, _C), lambda i: (i, 0)),
            pl.BlockSpec((_R, _DW), lambda i: (i, 0)),
        ],
        out_specs=pl.BlockSpec((_R, _C), lambda i: (i, 0)),
        out_shape=jax.ShapeDtypeStruct((_N, _C), F32),
    )(pre3, acc_a, acc_b, invd)


def kernel(x, edge_index, W1s, W1n, b1, W2s, W2n, b2, W3s, W3n, b3):
    npad = _EP - _E
    src2d = jnp.concatenate(
        [edge_index[0], jnp.zeros((npad,), jnp.int32)]).reshape(_ROWS, _CH)
    dst2d = jnp.concatenate(
        [edge_index[1], jnp.full((npad,), _N, jnp.int32)]).reshape(_ROWS, _CH)

    a1a, a1b, dga, dgb = _edge_agg(x, src2d, dst2d, _D, with_deg=True)
    pre1 = _pre(x, W1s, b1, _H)
    h2, hn2, invd = _combine1(pre1, a1a, a1b, dga, dgb, W1n, W2n)
    a2a, a2b = _edge_agg(hn2, src2d, dst2d, _H)
    pre2 = _pre(h2, W2s, b2, _H)
    h3, hn3 = _combine2(pre2, a2a, a2b, invd, W3n)
    a3a, a3b = _edge_agg(hn3, src2d, dst2d, _C)
    pre3 = _pre(h3, W3s, b3, _C)
    return _final(pre3, a3a, a3b, invd)


# speedup vs baseline: 2.0267x; 1.8369x over previous
import jax
import jax.numpy as jnp
from jax import lax
from jax.experimental import pallas as pl
from jax.experimental.pallas import tpu as pltpu
from jax.experimental.pallas import tpu_sc as plsc

F32 = jnp.float32

_N = 10000
_E = 320000
_D = 128
_H = 128
_C = 64

_CH = 64
_NSC = 2
_NTS = 16
_NW = _NSC * _NTS
_RPW = 158
_ROWS = _RPW * _NW
_EP = _ROWS * _CH
_SPILL = 64
_BPT = 624
_LAST = _N - _BPT * _NTS
_DW = 16

_R = 1000
_G = _N // _R


def _edge_agg(table, src2d, dst2d, W, with_deg=False):
    mesh = plsc.VectorSubcoreMesh(core_axis_name="c", subcore_axis_name="s")

    def body(tbl, src_h, dst_h, *rest):
        if with_deg:
            (out_a, out_b, deg_a, deg_b,
             src_v, dst_v, rows_v, rows_w, ones_v, acc, dacc, sem, semb) = rest
        else:
            (out_a, out_b,
             src_v, dst_v, rows_v, rows_w, acc, sem, semb) = rest
        c = lax.axis_index("c")
        s = lax.axis_index("s")
        wid = c * _NTS + s

        def zb(i, carry):
            r = i // (W // 16)
            col = (i % (W // 16)) * 16
            rows_v[r, pl.ds(col, 16)] = jnp.zeros((16,), F32)
            return carry
        lax.fori_loop(0, _CH * (W // 16), zb, 0)
        row0 = s * _BPT
        for k in range(_BPT // _CH):
            pltpu.sync_copy(rows_v, acc.at[pl.ds(row0 + k * _CH, _CH)])
        if _BPT % _CH:
            pltpu.sync_copy(rows_v.at[pl.ds(0, _BPT % _CH)],
                            acc.at[pl.ds(row0 + (_BPT // _CH) * _CH, _BPT % _CH)])

        @pl.when(s == _NTS - 1)
        def _():
            pltpu.sync_copy(rows_v.at[pl.ds(0, _LAST)],
                            acc.at[pl.ds(_NTS * _BPT, _LAST)])

        if with_deg:
            def zo(i, carry):
                ones_v[i, pl.ds(0, _DW)] = jnp.zeros((_DW,), F32)
                return carry
            lax.fori_loop(0, _CH, zo, 0)
            for k in range(_BPT // _CH):
                pltpu.sync_copy(ones_v, dacc.at[pl.ds(row0 + k * _CH, _CH)])
            if _BPT % _CH:
                pltpu.sync_copy(ones_v.at[pl.ds(0, _BPT % _CH)],
                                dacc.at[pl.ds(row0 + (_BPT // _CH) * _CH,
                                              _BPT % _CH)])

            @pl.when(s == _NTS - 1)
            def _():
                pltpu.sync_copy(ones_v.at[pl.ds(0, _LAST)],
                                dacc.at[pl.ds(_NTS * _BPT, _LAST)])

            def fo(i, carry):
                ones_v[i, pl.ds(0, _DW)] = jnp.full((_DW,), 1.0, F32)
                return carry
            lax.fori_loop(0, _CH, fo, 0)
        plsc.subcore_barrier()

        rb = wid * _RPW
        pltpu.sync_copy(src_h.at[pl.ds(rb, _RPW)], src_v)
        pltpu.sync_copy(dst_h.at[pl.ds(rb, _RPW)], dst_v)

        pltpu.async_copy(tbl.at[src_v.at[0]], rows_v, sem)

        def chunk(jj, carry):
            j = 2 * jj
            cpb = pltpu.async_copy(tbl.at[src_v.at[j + 1]], rows_w, semb)
            pltpu.make_async_copy(tbl.at[src_v.at[j]], rows_v, sem).wait()
            pltpu.sync_copy(rows_v, acc.at[dst_v.at[j]], add=True)
            if with_deg:
                pltpu.sync_copy(ones_v, dacc.at[dst_v.at[j]], add=True)

            @pl.when(jj < _RPW // 2 - 1)
            def _():
                pltpu.async_copy(tbl.at[src_v.at[j + 2]], rows_v, sem)
            cpb.wait()
            pltpu.sync_copy(rows_w, acc.at[dst_v.at[j + 1]], add=True)
            if with_deg:
                pltpu.sync_copy(ones_v, dacc.at[dst_v.at[j + 1]], add=True)
            return carry
        lax.fori_loop(0, _RPW // 2, chunk, 0)
        plsc.subcore_barrier()

        def copy_out(dst_hbm, deg_hbm):
            pltpu.sync_copy(acc.at[pl.ds(row0, _BPT)],
                            dst_hbm.at[pl.ds(row0, _BPT)])
            if with_deg:
                pltpu.sync_copy(dacc.at[pl.ds(row0, _BPT)],
                                deg_hbm.at[pl.ds(row0, _BPT)])

            @pl.when(s == _NTS - 1)
            def _():
                pltpu.sync_copy(acc.at[pl.ds(_NTS * _BPT, _LAST)],
                                dst_hbm.at[pl.ds(_NTS * _BPT, _LAST)])
                if with_deg:
                    pltpu.sync_copy(dacc.at[pl.ds(_NTS * _BPT, _LAST)],
                                    deg_hbm.at[pl.ds(_NTS * _BPT, _LAST)])

        @pl.when(c == 0)
        def _():
            copy_out(out_a, deg_a if with_deg else None)

        @pl.when(c == 1)
        def _():
            copy_out(out_b, deg_b if with_deg else None)

    out = jax.ShapeDtypeStruct((_N, W), F32)
    deg = jax.ShapeDtypeStruct((_N, _DW), F32)
    out_type = (out, out, deg, deg) if with_deg else (out, out)
    scratch = [
        pltpu.VMEM((_RPW, _CH), jnp.int32),
        pltpu.VMEM((_RPW, _CH), jnp.int32),
        pltpu.VMEM((_CH, W), F32),
        pltpu.VMEM((_CH, W), F32),
    ]
    if with_deg:
        scratch.append(pltpu.VMEM((_CH, _DW), F32))
    scratch.append(pltpu.VMEM_SHARED((_N + _SPILL, W), F32))
    if with_deg:
        scratch.append(pltpu.VMEM_SHARED((_N + _SPILL, _DW), F32))
    scratch += [pltpu.SemaphoreType.DMA, pltpu.SemaphoreType.DMA]
    k = pl.kernel(
        body,
        out_type=out_type,
        mesh=mesh,
        compiler_params=pltpu.CompilerParams(use_tc_tiling_on_sc=False),
        scratch_types=scratch,
    )
    return k(table, src2d, dst2d)


def _pre(h, w, b, dout):
    din = h.shape[1]

    def body(h_ref, w_ref, b_ref, o_ref):
        o_ref[...] = (jnp.dot(h_ref[...], w_ref[...], preferred_element_type=F32)
                      + b_ref[...])

    return pl.pallas_call(
        body,
        grid=(_G,),
        in_specs=[
            pl.BlockSpec((_R, din), lambda i: (i, 0)),
            pl.BlockSpec((din, dout), lambda i: (0, 0)),
            pl.BlockSpec((1, dout), lambda i: (0, 0)),
        ],
        out_specs=pl.BlockSpec((_R, dout), lambda i: (i, 0)),
        out_shape=jax.ShapeDtypeStruct((_N, dout), F32),
    )(h, w, b.reshape(1, dout))


def _combine1(pre1, acc_a, acc_b, deg_a, deg_b, w1n, w2n):
    def body(p_ref, aa_ref, ab_ref, da_ref, db_ref, wn_ref, w2_ref,
             h2_ref, hn2_ref, iv_ref):
        deg = da_ref[...] + db_ref[...]
        invd = 1.0 / jnp.maximum(deg, 1.0)
        mean = (aa_ref[...] + ab_ref[...]) * invd[:, 0:1]
        h2 = jnp.maximum(
            p_ref[...] + jnp.dot(mean, wn_ref[...], preferred_element_type=F32),
            0.0)
        h2_ref[...] = h2
        hn2_ref[...] = jnp.dot(h2, w2_ref[...], preferred_element_type=F32)
        iv_ref[...] = invd

    return pl.pallas_call(
        body,
        grid=(_G,),
        in_specs=[
            pl.BlockSpec((_R, _H), lambda i: (i, 0)),
            pl.BlockSpec((_R, _D), lambda i: (i, 0)),
            pl.BlockSpec((_R, _D), lambda i: (i, 0)),
            pl.BlockSpec((_R, _DW), lambda i: (i, 0)),
            pl.BlockSpec((_R, _DW), lambda i: (i, 0)),
            pl.BlockSpec((_D, _H), lambda i: (0, 0)),
            pl.BlockSpec((_H, _H), lambda i: (0, 0)),
        ],
        out_specs=[
            pl.BlockSpec((_R, _H), lambda i: (i, 0)),
            pl.BlockSpec((_R, _H), lambda i: (i, 0)),
            pl.BlockSpec((_R, _DW), lambda i: (i, 0)),
        ],
        out_shape=[
            jax.ShapeDtypeStruct((_N, _H), F32),
            jax.ShapeDtypeStruct((_N, _H), F32),
            jax.ShapeDtypeStruct((_N, _DW), F32),
        ],
    )(pre1, acc_a, acc_b, deg_a, deg_b, w1n, w2n)


def _combine2(pre2, acc_a, acc_b, invd, w3n):
    def body(p_ref, aa_ref, ab_ref, iv_ref, wn_ref, h3_ref, hn3_ref):
        mean = (aa_ref[...] + ab_ref[...]) * iv_ref[...][:, 0:1]
        h3 = jnp.maximum(p_ref[...] + mean, 0.0)
        h3_ref[...] = h3
        hn3_ref[...] = jnp.dot(h3, wn_ref[...], preferred_element_type=F32)

    return pl.pallas_call(
        body,
        grid=(_G,),
        in_specs=[
            pl.BlockSpec((_R, _H), lambda i: (i, 0)),
            pl.BlockSpec((_R, _H), lambda i: (i, 0)),
            pl.BlockSpec((_R, _H), lambda i: (i, 0)),
            pl.BlockSpec((_R, _DW), lambda i: (i, 0)),
            pl.BlockSpec((_H, _C), lambda i: (0, 0)),
        ],
        out_specs=[
            pl.BlockSpec((_R, _H), lambda i: (i, 0)),
            pl.BlockSpec((_R, _C), lambda i: (i, 0)),
        ],
        out_shape=[
            jax.ShapeDtypeStruct((_N, _H), F32),
            jax.ShapeDtypeStruct((_N, _C), F32),
        ],
    )(pre2, acc_a, acc_b, invd, w3n)


def _final(pre3, acc_a, acc_b, invd):
    def body(p_ref, aa_ref, ab_ref, iv_ref, o_ref):
        mean = (aa_ref[...] + ab_ref[...]) * iv_ref[...][:, 0:1]
        z = jnp.maximum(p_ref[...] + mean, 0.0)
        m = jnp.max(z, axis=1, keepdims=True)
        lse = jnp.log(jnp.sum(jnp.exp(z - m), axis=1, keepdims=True)) + m
        o_ref[...] = z - lse

    return pl.pallas_call(
        body,
        grid=(_G,),
        in_specs=[
            pl.BlockSpec((_R, _C), lambda i: (i, 0)),
            pl.BlockSpec((_R, _C), lambda i: (i, 0)),
            pl.BlockSpec((_R, _C), lambda i: (i, 0)),
            pl.BlockSpec((_R, _DW), lambda i: (i, 0)),
        ],
        out_specs=pl.BlockSpec((_R, _C), lambda i: (i, 0)),
        out_shape=jax.ShapeDtypeStruct((_N, _C), F32),
    )(pre3, acc_a, acc_b, invd)


def kernel(x, edge_index, W1s, W1n, b1, W2s, W2n, b2, W3s, W3n, b3):
    npad = _EP - _E
    pad_iota = jnp.arange(npad, dtype=jnp.int32)
    src2d = jnp.concatenate(
        [edge_index[0], pad_iota % _N]).reshape(_ROWS, _CH)
    dst2d = jnp.concatenate(
        [edge_index[1], _N + pad_iota % _SPILL]).reshape(_ROWS, _CH)

    a1a, a1b, dga, dgb = _edge_agg(x, src2d, dst2d, _D, with_deg=True)
    pre1 = _pre(x, W1s, b1, _H)
    h2, hn2, invd = _combine1(pre1, a1a, a1b, dga, dgb, W1n, W2n)
    a2a, a2b = _edge_agg(hn2, src2d, dst2d, _H)
    pre2 = _pre(h2, W2s, b2, _H)
    h3, hn3 = _combine2(pre2, a2a, a2b, invd, W3n)
    a3a, a3b = _edge_agg(hn3, src2d, dst2d, _C)
    pre3 = _pre(h3, W3s, b3, _C)
    return _final(pre3, a3a, a3b, invd)


# speedup vs baseline: 2.0739x; 1.0233x over previous
import jax
import jax.numpy as jnp
from jax import lax
from jax.experimental import pallas as pl
from jax.experimental.pallas import tpu as pltpu
from jax.experimental.pallas import tpu_sc as plsc

F32 = jnp.float32

_N = 10000
_E = 320000
_D = 128
_H = 128
_C = 64

_CH = 64
_NSC = 2
_NTS = 16
_NW = _NSC * _NTS
_ROWS = _E // _CH
_RPW = 158
_RLAST = _ROWS - _RPW * (_NW - 1)
_BPT = 624
_LAST = _N - _BPT * _NTS
_DW = 16

_R = 1000
_G = _N // _R


def _edge_agg(table, edges3d, W, with_deg=False):
    mesh = plsc.VectorSubcoreMesh(core_axis_name="c", subcore_axis_name="s")

    def body(tbl, edges_h, *rest):
        if with_deg:
            (out_a, out_b, deg_a, deg_b,
             src_v, dst_v, rows_v, rows_w, ones_v, acc, dacc, sem, semb) = rest
        else:
            (out_a, out_b,
             src_v, dst_v, rows_v, rows_w, acc, sem, semb) = rest
        c = lax.axis_index("c")
        s = lax.axis_index("s")
        wid = c * _NTS + s

        def zb(i, carry):
            r = i // (W // 16)
            col = (i % (W // 16)) * 16
            rows_v[r, pl.ds(col, 16)] = jnp.zeros((16,), F32)
            return carry
        lax.fori_loop(0, _CH * (W // 16), zb, 0)
        row0 = s * _BPT
        for k in range(_BPT // _CH):
            pltpu.sync_copy(rows_v, acc.at[pl.ds(row0 + k * _CH, _CH)])
        if _BPT % _CH:
            pltpu.sync_copy(rows_v.at[pl.ds(0, _BPT % _CH)],
                            acc.at[pl.ds(row0 + (_BPT // _CH) * _CH, _BPT % _CH)])

        @pl.when(s == _NTS - 1)
        def _():
            pltpu.sync_copy(rows_v.at[pl.ds(0, _LAST)],
                            acc.at[pl.ds(_NTS * _BPT, _LAST)])

        if with_deg:
            def zo(i, carry):
                ones_v[i, pl.ds(0, _DW)] = jnp.zeros((_DW,), F32)
                return carry
            lax.fori_loop(0, _CH, zo, 0)
            for k in range(_BPT // _CH):
                pltpu.sync_copy(ones_v, dacc.at[pl.ds(row0 + k * _CH, _CH)])
            if _BPT % _CH:
                pltpu.sync_copy(ones_v.at[pl.ds(0, _BPT % _CH)],
                                dacc.at[pl.ds(row0 + (_BPT // _CH) * _CH,
                                              _BPT % _CH)])

            @pl.when(s == _NTS - 1)
            def _():
                pltpu.sync_copy(ones_v.at[pl.ds(0, _LAST)],
                                dacc.at[pl.ds(_NTS * _BPT, _LAST)])

            def fo(i, carry):
                ones_v[i, pl.ds(0, _DW)] = jnp.full((_DW,), 1.0, F32)
                return carry
            lax.fori_loop(0, _CH, fo, 0)
        plsc.subcore_barrier()

        rb = wid * _RPW

        @pl.when(wid < _NW - 1)
        def _():
            pltpu.sync_copy(edges_h.at[0, pl.ds(rb, _RPW)], src_v)
            pltpu.sync_copy(edges_h.at[1, pl.ds(rb, _RPW)], dst_v)

        @pl.when(wid == _NW - 1)
        def _():
            pltpu.sync_copy(edges_h.at[0, pl.ds(rb, _RLAST)],
                            src_v.at[pl.ds(0, _RLAST)])
            pltpu.sync_copy(edges_h.at[1, pl.ds(rb, _RLAST)],
                            dst_v.at[pl.ds(0, _RLAST)])

        niter = jnp.where(wid == _NW - 1, _RLAST // 2, _RPW // 2)

        pltpu.async_copy(tbl.at[src_v.at[0]], rows_v, sem)

        def chunk(jj, carry):
            j = 2 * jj
            cpb = pltpu.async_copy(tbl.at[src_v.at[j + 1]], rows_w, semb)
            pltpu.make_async_copy(tbl.at[src_v.at[j]], rows_v, sem).wait()
            pltpu.sync_copy(rows_v, acc.at[dst_v.at[j]], add=True)
            if with_deg:
                pltpu.sync_copy(ones_v, dacc.at[dst_v.at[j]], add=True)

            @pl.when(jj < niter - 1)
            def _():
                pltpu.async_copy(tbl.at[src_v.at[j + 2]], rows_v, sem)
            cpb.wait()
            pltpu.sync_copy(rows_w, acc.at[dst_v.at[j + 1]], add=True)
            if with_deg:
                pltpu.sync_copy(ones_v, dacc.at[dst_v.at[j + 1]], add=True)
            return carry
        lax.fori_loop(0, niter, chunk, 0)
        plsc.subcore_barrier()

        def copy_out(dst_hbm, deg_hbm):
            pltpu.sync_copy(acc.at[pl.ds(row0, _BPT)],
                            dst_hbm.at[pl.ds(row0, _BPT)])
            if with_deg:
                pltpu.sync_copy(dacc.at[pl.ds(row0, _BPT)],
                                deg_hbm.at[pl.ds(row0, _BPT)])

            @pl.when(s == _NTS - 1)
            def _():
                pltpu.sync_copy(acc.at[pl.ds(_NTS * _BPT, _LAST)],
                                dst_hbm.at[pl.ds(_NTS * _BPT, _LAST)])
                if with_deg:
                    pltpu.sync_copy(dacc.at[pl.ds(_NTS * _BPT, _LAST)],
                                    deg_hbm.at[pl.ds(_NTS * _BPT, _LAST)])

        @pl.when(c == 0)
        def _():
            copy_out(out_a, deg_a if with_deg else None)

        @pl.when(c == 1)
        def _():
            copy_out(out_b, deg_b if with_deg else None)

    out = jax.ShapeDtypeStruct((_N, W), F32)
    deg = jax.ShapeDtypeStruct((_N, _DW), F32)
    out_type = (out, out, deg, deg) if with_deg else (out, out)
    scratch = [
        pltpu.VMEM((_RPW, _CH), jnp.int32),
        pltpu.VMEM((_RPW, _CH), jnp.int32),
        pltpu.VMEM((_CH, W), F32),
        pltpu.VMEM((_CH, W), F32),
    ]
    if with_deg:
        scratch.append(pltpu.VMEM((_CH, _DW), F32))
    scratch.append(pltpu.VMEM_SHARED((_N, W), F32))
    if with_deg:
        scratch.append(pltpu.VMEM_SHARED((_N, _DW), F32))
    scratch += [pltpu.SemaphoreType.DMA, pltpu.SemaphoreType.DMA]
    k = pl.kernel(
        body,
        out_type=out_type,
        mesh=mesh,
        compiler_params=pltpu.CompilerParams(use_tc_tiling_on_sc=False),
        scratch_types=scratch,
    )
    return k(table, edges3d)


def _pre(h, w, b, dout):
    din = h.shape[1]

    def body(h_ref, w_ref, b_ref, o_ref):
        o_ref[...] = (jnp.dot(h_ref[...], w_ref[...], preferred_element_type=F32)
                      + b_ref[...])

    return pl.pallas_call(
        body,
        grid=(_G,),
        in_specs=[
            pl.BlockSpec((_R, din), lambda i: (i, 0)),
            pl.BlockSpec((din, dout), lambda i: (0, 0)),
            pl.BlockSpec((1, dout), lambda i: (0, 0)),
        ],
        out_specs=pl.BlockSpec((_R, dout), lambda i: (i, 0)),
        out_shape=jax.ShapeDtypeStruct((_N, dout), F32),
    )(h, w, b.reshape(1, dout))


def _combine1(pre1, acc_a, acc_b, deg_a, deg_b, w1n, w2n):
    def body(p_ref, aa_ref, ab_ref, da_ref, db_ref, wn_ref, w2_ref,
             h2_ref, hn2_ref, iv_ref):
        deg = da_ref[...] + db_ref[...]
        invd = 1.0 / jnp.maximum(deg, 1.0)
        mean = (aa_ref[...] + ab_ref[...]) * invd[:, 0:1]
        h2 = jnp.maximum(
            p_ref[...] + jnp.dot(mean, wn_ref[...], preferred_element_type=F32),
            0.0)
        h2_ref[...] = h2
        hn2_ref[...] = jnp.dot(h2, w2_ref[...], preferred_element_type=F32)
        iv_ref[...] = invd

    return pl.pallas_call(
        body,
        grid=(_G,),
        in_specs=[
            pl.BlockSpec((_R, _H), lambda i: (i, 0)),
            pl.BlockSpec((_R, _D), lambda i: (i, 0)),
            pl.BlockSpec((_R, _D), lambda i: (i, 0)),
            pl.BlockSpec((_R, _DW), lambda i: (i, 0)),
            pl.BlockSpec((_R, _DW), lambda i: (i, 0)),
            pl.BlockSpec((_D, _H), lambda i: (0, 0)),
            pl.BlockSpec((_H, _H), lambda i: (0, 0)),
        ],
        out_specs=[
            pl.BlockSpec((_R, _H), lambda i: (i, 0)),
            pl.BlockSpec((_R, _H), lambda i: (i, 0)),
            pl.BlockSpec((_R, _DW), lambda i: (i, 0)),
        ],
        out_shape=[
            jax.ShapeDtypeStruct((_N, _H), F32),
            jax.ShapeDtypeStruct((_N, _H), F32),
            jax.ShapeDtypeStruct((_N, _DW), F32),
        ],
    )(pre1, acc_a, acc_b, deg_a, deg_b, w1n, w2n)


def _combine2(pre2, acc_a, acc_b, invd, w3n):
    def body(p_ref, aa_ref, ab_ref, iv_ref, wn_ref, h3_ref, hn3_ref):
        mean = (aa_ref[...] + ab_ref[...]) * iv_ref[...][:, 0:1]
        h3 = jnp.maximum(p_ref[...] + mean, 0.0)
        h3_ref[...] = h3
        hn3_ref[...] = jnp.dot(h3, wn_ref[...], preferred_element_type=F32)

    return pl.pallas_call(
        body,
        grid=(_G,),
        in_specs=[
            pl.BlockSpec((_R, _H), lambda i: (i, 0)),
            pl.BlockSpec((_R, _H), lambda i: (i, 0)),
            pl.BlockSpec((_R, _H), lambda i: (i, 0)),
            pl.BlockSpec((_R, _DW), lambda i: (i, 0)),
            pl.BlockSpec((_H, _C), lambda i: (0, 0)),
        ],
        out_specs=[
            pl.BlockSpec((_R, _H), lambda i: (i, 0)),
            pl.BlockSpec((_R, _C), lambda i: (i, 0)),
        ],
        out_shape=[
            jax.ShapeDtypeStruct((_N, _H), F32),
            jax.ShapeDtypeStruct((_N, _C), F32),
        ],
    )(pre2, acc_a, acc_b, invd, w3n)


def _final(pre3, acc_a, acc_b, invd):
    def body(p_ref, aa_ref, ab_ref, iv_ref, o_ref):
        mean = (aa_ref[...] + ab_ref[...]) * iv_ref[...][:, 0:1]
        z = jnp.maximum(p_ref[...] + mean, 0.0)
        m = jnp.max(z, axis=1, keepdims=True)
        lse = jnp.log(jnp.sum(jnp.exp(z - m), axis=1, keepdims=True)) + m
        o_ref[...] = z - lse

    return pl.pallas_call(
        body,
        grid=(_G,),
        in_specs=[
            pl.BlockSpec((_R, _C), lambda i: (i, 0)),
            pl.BlockSpec((_R, _C), lambda i: (i, 0)),
            pl.BlockSpec((_R, _C), lambda i: (i, 0)),
            pl.BlockSpec((_R, _DW), lambda i: (i, 0)),
        ],
        out_specs=pl.BlockSpec((_R, _C), lambda i: (i, 0)),
        out_shape=jax.ShapeDtypeStruct((_N, _C), F32),
    )(pre3, acc_a, acc_b, invd)


def kernel(x, edge_index, W1s, W1n, b1, W2s, W2n, b2, W3s, W3n, b3):
    edges3d = edge_index.reshape(2, _ROWS, _CH)

    a1a, a1b, dga, dgb = _edge_agg(x, edges3d, _D, with_deg=True)
    pre1 = _pre(x, W1s, b1, _H)
    h2, hn2, invd = _combine1(pre1, a1a, a1b, dga, dgb, W1n, W2n)
    a2a, a2b = _edge_agg(hn2, edges3d, _H)
    pre2 = _pre(h2, W2s, b2, _H)
    h3, hn3 = _combine2(pre2, a2a, a2b, invd, W3n)
    a3a, a3b = _edge_agg(hn3, edges3d, _C)
    pre3 = _pre(h3, W3s, b3, _C)
    return _final(pre3, a3a, a3b, invd)
